# Initial kernel scaffold; baseline (speedup 1.0000x reference)
import jax
import jax.numpy as jnp
from jax.experimental import pallas as pl


def kernel(x, edge_index, edge_weigth, batch, W1, b1, g1, be1, p1rw, p1rb, p1ow, W2, b2, g2, be2, p2rw, p2rb, p2ow, f1w, f1b, f2w, f2b, f3w, f3b):
    raise NotImplementedError("write your pallas kernel here")


# speedup vs baseline: 29.4237x; 29.4237x over previous
import functools

import jax
import jax.numpy as jnp
from jax import lax
from jax.experimental import pallas as pl
from jax.experimental.pallas import tpu as pltpu
from jax.experimental.pallas import tpu_sc as plsc

_NSUB = 16
_NWORKERS = 32
_CE = 10240


@functools.lru_cache(maxsize=None)
def _edge_pass_kernel(n, d, e_pad):
    wpc = _NWORKERS // d
    ew = e_pad // wpc
    nchunks = ew // _CE
    n_pad = n + _NSUB
    mesh = plsc.VectorSubcoreMesh(core_axis_name="c", subcore_axis_name="s")

    @functools.partial(
        pl.kernel,
        out_type=jax.ShapeDtypeStruct((_NWORKERS * n,), jnp.float32),
        mesh=mesh,
        compiler_params=pltpu.CompilerParams(needs_layout_passes=False),
        scratch_types=[
            pltpu.VMEM((n_pad,), jnp.float32),
            pltpu.VMEM((n_pad,), jnp.float32),
            pltpu.VMEM((_CE,), jnp.int32),
            pltpu.VMEM((_CE,), jnp.int32),
        ],
    )
    def kern(tflat_hbm, zeros_hbm, src_hbm, dst_hbm, out_hbm, tv, av, sv, dv):
        c = lax.axis_index("c")
        s = lax.axis_index("s")
        wid = c * _NSUB + s
        col = wid % d
        shard = wid // d

        pltpu.sync_copy(tflat_hbm.at[pl.ds(pl.multiple_of(col * n, 8), n)],
                        tv.at[pl.ds(0, n)])
        pltpu.sync_copy(zeros_hbm, av.at[pl.ds(0, n)])

        def chunk_body(k, carry):
            off = pl.multiple_of(shard * ew + k * _CE, 8)
            pltpu.sync_copy(src_hbm.at[pl.ds(off, _CE)], sv)
            pltpu.sync_copy(dst_hbm.at[pl.ds(off, _CE)], dv)

            def gbody(g, carry2):
                si = sv[pl.ds(g * 16, 16)]
                di = dv[pl.ds(g * 16, 16)]
                vals = plsc.load_gather(tv, [si])
                plsc.addupdate_scatter(av, [di], vals)
                return carry2

            return lax.fori_loop(0, _CE // 16, gbody, carry)

        lax.fori_loop(0, nchunks, chunk_body, 0)
        pltpu.sync_copy(av.at[pl.ds(0, n)],
                        out_hbm.at[pl.ds(pl.multiple_of(wid * n, 8), n)])

    return kern


def _edge_pass(n, d, e_pad):
    kern = _edge_pass_kernel(n, d, e_pad)
    wpc = _NWORKERS // d

    def run(table_t_flat, zeros, src_flat, dst_flat):
        out = kern(table_t_flat, zeros, src_flat, dst_flat)
        return out.reshape(wpc, d, n)

    return run


def _leaky(v):
    return jnp.where(v >= 0, v, 0.01 * v)


def _dgT(a, b):
    return lax.dot_general(a, b, (((0,), (0,)), ((), ())),
                           preferred_element_type=jnp.float32)


def _mm_k(x_ref, w_ref, o_ref):
    o_ref[...] = lax.dot_general(w_ref[...], x_ref[...],
                                 (((0,), (1,)), ((), ())),
                                 preferred_element_type=jnp.float32)


def _bn1_k(aggp_ref, b_ref, g_ref, be_ref, rw_ref, rb_ref, ow_ref,
           h_ref, hs_ref, ho_ref):
    v = jnp.sum(aggp_ref[...], axis=0) + b_ref[...][:, None]
    m = jnp.mean(v, axis=1, keepdims=True)
    var = jnp.mean(v * v, axis=1, keepdims=True) - m * m
    h = _leaky((v - m) * (g_ref[...][:, None] * lax.rsqrt(var + 1e-5))
               + be_ref[...][:, None])
    h_ref[...] = h
    hs_ref[...] = _dgT(rw_ref[...], h)
    ho_ref[...] = _dgT(ow_ref[...], h) + rb_ref[...][:, None]


def _mono_key(score):
    u = lax.bitcast_convert_type(score, jnp.uint32)
    return u ^ jnp.where(u >> 31 != 0,
                         jnp.uint32(0xFFFFFFFF), jnp.uint32(0x80000000))


def _topk_mask(score, valid, k):
    n = score.shape[1]
    key = _mono_key(score)

    def vbit(b, t):
        tt = t | (jnp.uint32(1) << b)
        cnt = jnp.sum(valid & (key >= tt))
        return jnp.where(cnt >= k, tt, t)

    t = lax.fori_loop(0, 32, lambda i, t: vbit(jnp.uint32(31 - i), t),
                      jnp.uint32(0))
    cnt_gt = jnp.sum(valid & (key > t))
    need = k - cnt_gt
    idx = lax.broadcasted_iota(jnp.int32, (1, n), 1)
    ties = valid & (key == t)

    def ibit(b, m):
        mm = m | (jnp.int32(1) << b)
        cnt = jnp.sum(ties & (idx < mm))
        return jnp.where(cnt < need, mm, m)

    m = lax.fori_loop(0, 15, lambda i, m: ibit(jnp.int32(14 - i), m),
                      jnp.int32(0))
    return valid & ((key > t) | (ties & (idx <= m)))


def _topk1_k(saggp_ref, ho_ref, h_ref, w2_ref, h2lin_ref, sel_ref, *, k):
    score = jnp.sum(saggp_ref[...], axis=0) + ho_ref[...]
    sel = _topk_mask(score, jnp.full(score.shape, True), k)
    xn = jnp.where(sel, h_ref[...] * jnp.tanh(score), 0.0)
    h2lin_ref[...] = _dgT(w2_ref[...], xn)
    sel_ref[...] = sel.astype(jnp.float32)


def _bn2_k(aggp_ref, sel_ref, b_ref, g_ref, be_ref, rw_ref, rb_ref, ow_ref,
           h2_ref, hs_ref, ho_ref, *, k):
    selv = sel_ref[...]
    v = jnp.sum(aggp_ref[...], axis=0) + b_ref[...][:, None]
    m = jnp.sum(v * selv, axis=1, keepdims=True) / k
    var = jnp.sum(v * v * selv, axis=1, keepdims=True) / k - m * m
    h2 = selv * _leaky((v - m) * (g_ref[...][:, None] * lax.rsqrt(var + 1e-5))
                       + be_ref[...][:, None])
    h2_ref[...] = h2
    hs_ref[...] = _dgT(rw_ref[...], h2)
    ho_ref[...] = _dgT(ow_ref[...], h2) + rb_ref[...][:, None]


def _final_k(saggp_ref, ho_ref, sel_ref, h2_ref,
             f1w_ref, f1b_ref, f2w_ref, f2b_ref, f3w_ref, f3b_ref,
             o_ref, *, k):
    score = jnp.sum(saggp_ref[...], axis=0) + ho_ref[...]
    sel2 = _topk_mask(score, sel_ref[...] > 0, k)
    w = jnp.where(sel2, jnp.tanh(score), 0.0)
    ap = jnp.sum(h2_ref[...] * w, axis=1, keepdims=True)
    o = _leaky(_dgT(f1w_ref[...], ap) + f1b_ref[...][:, None])
    o = _leaky(_dgT(f2w_ref[...], o) + f2b_ref[...][:, None])
    o = _leaky(_dgT(f3w_ref[...], o) + f3b_ref[...][:, None])
    o_ref[...] = o


def _call(body, out_shapes):
    return pl.pallas_call(
        body, out_shape=[jax.ShapeDtypeStruct(s, jnp.float32)
                         for s in out_shapes])


def kernel(x, edge_index, edge_weigth, batch,
           W1, b1, g1, be1, p1rw, p1rb, p1ow,
           W2, b2, g2, be2, p2rw, p2rb, p2ow,
           f1w, f1b, f2w, f2b, f3w, f3b):
    n, _ = x.shape
    e = edge_index.shape[1]
    k1 = -(-n // 2)
    k2 = -(-k1 // 2)
    d1 = W1.shape[1]
    d2 = W2.shape[1]

    e_pad = -(-e // (_NWORKERS * _CE)) * (_NWORKERS * _CE)
    pad_idx = n + (jnp.arange(e_pad - e, dtype=jnp.int32) % _NSUB)
    src = jnp.concatenate([edge_index[0], pad_idx])
    dst = jnp.concatenate([edge_index[1], pad_idx])

    ep16 = _edge_pass(n, d1, e_pad)
    ep1 = _edge_pass(n, 1, e_pad)
    ep32 = _edge_pass(n, d2, e_pad)
    zeros = jnp.zeros((n,), jnp.float32)

    (hlinT,) = _call(_mm_k, [(d1, n)])(x, W1)
    agg1p = ep16(hlinT.reshape(-1), zeros, src, dst)
    hT, hs1, ho1 = _call(_bn1_k, [(d1, n), (1, n), (1, n)])(
        agg1p, b1, g1, be1, p1rw, p1rb, p1ow)

    sagg1p = ep1(hs1.reshape(-1), zeros, src, dst)
    h2linT, sel1 = _call(functools.partial(_topk1_k, k=k1),
                         [(d2, n), (1, n)])(sagg1p, ho1, hT, W2)

    agg2p = ep32(h2linT.reshape(-1), zeros, src, dst)
    h2T, hs2, ho2 = _call(functools.partial(_bn2_k, k=k1),
                          [(d2, n), (1, n), (1, n)])(
        agg2p, sel1, b2, g2, be2, p2rw, p2rb, p2ow)

    sagg2p = ep1(hs2.reshape(-1), zeros, src, dst)
    (o,) = _call(functools.partial(_final_k, k=k2), [(2, 1)])(
        sagg2p, ho2, sel1, h2T, f1w, f1b, f2w, f2b, f3w, f3b)
    return o.T
